# baseline (speedup 1.0000x reference)
import functools

import jax
import jax.numpy as jnp
from jax import lax
from jax.experimental import pallas as pl
from jax.experimental.pallas import tpu as pltpu
from jax.experimental.pallas import tpu_sc as plsc


def _sc_call(bs, nf, h, w):
    mesh = plsc.VectorSubcoreMesh(core_axis_name="c", subcore_axis_name="s")

    @functools.partial(
        pl.kernel,
        mesh=mesh,
        out_type=jax.ShapeDtypeStruct((bs, h, w, 2 * nf), jnp.float32),
        scratch_types=[
            pltpu.VMEM((8, w, 2 * nf), jnp.float32),
            pltpu.VMEM((8, nf), jnp.float32),
            pltpu.SemaphoreType.DMA,
        ],
    )
    def k(col_hbm, row_hbm, out_hbm, buf_v, rows_v, sem):
        cid = lax.axis_index("c")
        sid = lax.axis_index("s")
        wid = sid * 2 + cid
        b = wid // 2
        i0 = (wid % 2) * 16

        def build_chunk(r0):
            for kk in range(8):
                pltpu.sync_copy(
                    col_hbm.at[pl.ds(0, w)], buf_v.at[kk, :, pl.ds(0, nf)]
                )
            pltpu.sync_copy(row_hbm.at[pl.ds(r0, 8)], rows_v)
            for kk in range(8):
                vecs = [rows_v[kk, pl.ds(16 * t, 16)] for t in range(nf // 16)]

                def store_j(j, _):
                    for t, v in enumerate(vecs):
                        buf_v[kk, j, pl.ds(nf + 16 * t, 16)] = v
                    return 0

                lax.fori_loop(0, w, store_j, 0)

        build_chunk(i0)
        c0 = pltpu.make_async_copy(buf_v, out_hbm.at[b, pl.ds(i0, 8)], sem)
        c0.start()
        c0.wait()
        build_chunk(i0 + 8)
        c1 = pltpu.make_async_copy(
            buf_v, out_hbm.at[b, pl.ds(i0 + 8, 8)], sem
        )
        c1.start()
        c1.wait()

    return k


def kernel(mask, feature_map, row_embed, col_embed):
    h, w = mask.shape[-2], mask.shape[-1]
    bs = mask.shape[0]
    nf = row_embed.shape[1]
    q = _sc_call(bs, nf, h, w)(col_embed, row_embed)
    return jnp.transpose(q, (0, 3, 1, 2))


# speedup vs baseline: 7.6162x; 7.6162x over previous
import jax
import jax.numpy as jnp
from jax.experimental import pallas as pl
from jax.experimental.pallas import tpu as pltpu


def _pos_body(col_ref, row_ref, out_ref, scratch, sem):
    nf = col_ref.shape[1]
    h, w = scratch.shape[0], scratch.shape[1]
    bs = out_ref.shape[0]
    ce = col_ref[:w, :]
    re = row_ref[:h, :]
    scratch[:, :, :nf] = jnp.broadcast_to(ce[None, :, :], (h, w, nf))
    scratch[:, :, nf:] = jnp.broadcast_to(re[:, None, :], (h, w, nf))
    copies = [
        pltpu.make_async_copy(scratch, out_ref.at[b], sem) for b in range(bs)
    ]
    for c in copies:
        c.start()
    for c in copies:
        c.wait()


def kernel(mask, feature_map, row_embed, col_embed):
    h, w = mask.shape[-2], mask.shape[-1]
    bs = mask.shape[0]
    nf = row_embed.shape[1]
    q = pl.pallas_call(
        _pos_body,
        in_specs=[
            pl.BlockSpec(memory_space=pltpu.VMEM),
            pl.BlockSpec(memory_space=pltpu.VMEM),
        ],
        out_specs=pl.BlockSpec(memory_space=pl.ANY),
        out_shape=jax.ShapeDtypeStruct((bs, h, w, 2 * nf), jnp.float32),
        scratch_shapes=[
            pltpu.VMEM((h, w, 2 * nf), jnp.float32),
            pltpu.SemaphoreType.DMA,
        ],
    )(col_embed, row_embed)
    return jnp.transpose(q, (0, 3, 1, 2))
